# baseline (speedup 1.0000x reference)
import functools
import math

import jax
import jax.numpy as jnp
from jax import lax
from jax.experimental import pallas as pl
from jax.experimental.pallas import tpu as pltpu
from jax.experimental.pallas import tpu_sc as plsc

D_MODEL = 64
SCALE = math.sqrt(D_MODEL)


@functools.partial(jax.jit, static_argnames=("B", "D"))
def _emb_lookup(idx_flat, table, *, B, D):
    info = plsc.get_sparse_core_info()
    NC, NS, L = info.num_cores, info.num_subcores, info.num_lanes
    NW = NC * NS
    assert B % NW == 0
    b_per_w = B // NW
    C = 800
    n_chunks = b_per_w // C
    assert D % L == 0

    mesh = plsc.VectorSubcoreMesh(core_axis_name="c", subcore_axis_name="s")

    @functools.partial(
        pl.kernel,
        mesh=mesh,
        compiler_params=pltpu.CompilerParams(use_tc_tiling_on_sc=False),
        out_type=jax.ShapeDtypeStruct((B, D), jnp.float32),
        scratch_types=[
            pltpu.VMEM((b_per_w,), jnp.int32),
            pltpu.VMEM((C, D), jnp.float32),
            pltpu.SemaphoreType.DMA,
        ],
    )
    def k(idx_hbm, table_hbm, out_hbm, idx_v, rows_v, sem):
        wid = lax.axis_index("s") * NC + lax.axis_index("c")
        base = wid * b_per_w
        pltpu.sync_copy(idx_hbm.at[pl.ds(base, b_per_w)], idx_v)
        for j in range(n_chunks):
            pltpu.async_copy(
                table_hbm.at[idx_v.at[pl.ds(j * C, C)]], rows_v, sem
            ).wait()

            def scale_row(i, carry):
                for g in range(D // L):
                    sl = (i, pl.ds(g * L, L))
                    rows_v[sl] = rows_v[sl] * SCALE
                return carry

            lax.fori_loop(0, C, scale_row, 0)

            pltpu.sync_copy(rows_v, out_hbm.at[pl.ds(base + j * C, C)])

    return k(idx_flat, table)


def kernel(x, word_emb_weight):
    B = x.shape[0] * x.shape[1]
    D = word_emb_weight.shape[1]
    idx_flat = x.reshape(B)
    out = _emb_lookup(idx_flat, word_emb_weight, B=B, D=D)
    return out.reshape(x.shape[0], x.shape[1], D)


# speedup vs baseline: 1.3165x; 1.3165x over previous
import functools
import math

import jax
import jax.numpy as jnp
from jax import lax
from jax.experimental import pallas as pl
from jax.experimental.pallas import tpu as pltpu
from jax.experimental.pallas import tpu_sc as plsc

D_MODEL = 64
SCALE = math.sqrt(D_MODEL)


@functools.partial(jax.jit, static_argnames=("B", "D"))
def _emb_lookup(idx_flat, table, *, B, D):
    info = plsc.get_sparse_core_info()
    NC, NS, L = info.num_cores, info.num_subcores, info.num_lanes
    NW = NC * NS
    assert B % NW == 0
    b_per_w = B // NW
    C = 800
    n_chunks = b_per_w // C
    assert D % L == 0

    mesh = plsc.VectorSubcoreMesh(core_axis_name="c", subcore_axis_name="s")

    @functools.partial(
        pl.kernel,
        mesh=mesh,
        out_type=jax.ShapeDtypeStruct((B, D), jnp.float32),
        scratch_types=[
            pltpu.VMEM((b_per_w,), jnp.int32),
            pltpu.VMEM((C, D), jnp.float32),
            pltpu.SemaphoreType.DMA,
        ],
    )
    def k(idx_hbm, table_hbm, out_hbm, idx_v, rows_v, sem):
        wid = lax.axis_index("s") * NC + lax.axis_index("c")
        base = wid * b_per_w
        pltpu.sync_copy(idx_hbm.at[pl.ds(base, b_per_w)], idx_v)
        for j in range(n_chunks):
            def issue(r, carry):
                vec = idx_v[pl.ds(j * C + r * L, L)]
                for t in range(L):
                    pltpu.make_async_copy(
                        table_hbm.at[vec[t]], rows_v.at[r * L + t], sem
                    ).start()
                return carry

            lax.fori_loop(0, C // L, issue, 0)
            pltpu.make_async_copy(
                table_hbm.at[pl.ds(0, C)], rows_v, sem
            ).wait()

            def scale_row(i, carry):
                for g in range(D // L):
                    sl = (i, pl.ds(g * L, L))
                    rows_v[sl] = rows_v[sl] * SCALE
                return carry

            lax.fori_loop(0, C, scale_row, 0)

            pltpu.sync_copy(rows_v, out_hbm.at[pl.ds(base + j * C, C)])

    return k(idx_flat, table)


def kernel(x, word_emb_weight):
    B = x.shape[0] * x.shape[1]
    D = word_emb_weight.shape[1]
    idx_flat = x.reshape(B)
    out = _emb_lookup(idx_flat, word_emb_weight, B=B, D=D)
    return out.reshape(x.shape[0], x.shape[1], D)
